# baseline (device time: 45816 ns/iter reference)
import jax
import jax.numpy as jnp
from jax import lax
from jax.experimental import pallas as pl
from jax.experimental.pallas import tpu as pltpu

N_DEV = 8
B = 2
S = 256
D_MODEL = 512
HPB = 4
DH = 64
HB = HPB * DH

L_HOPS = N_DEV // 2
R_HOPS = N_DEV - 1 - L_HOPS


def kernel(x, Wq, K_ext, V_ext, Wo):
    K_t = jnp.transpose(K_ext, (0, 2, 1, 3)).astype(jnp.bfloat16)
    V_t = jnp.transpose(V_ext, (0, 2, 1, 3)).astype(jnp.bfloat16)
    x_b = x.astype(jnp.bfloat16)

    def body(x_ref, wq_ref, k_ref, v_ref, wo_ref, out_ref,
             wq_comm, wo_comm, swq, rwq, swo, rwo):
        my_pos = lax.axis_index("i")
        left = lax.rem(my_pos + N_DEV - 1, N_DEV)
        right = lax.rem(my_pos + 1, N_DEV)

        barrier_sem = pltpu.get_barrier_semaphore()
        for nbr in (left, right):
            pl.semaphore_signal(
                barrier_sem, inc=1,
                device_id=(nbr,), device_id_type=pl.DeviceIdType.MESH,
            )
        pl.semaphore_wait(barrier_sem, 2)

        wq_comm[0] = wq_ref[...].astype(jnp.bfloat16)
        wo_comm[0] = wo_ref[...].astype(jnp.bfloat16)

        def origin_of(slot):
            if slot == 0:
                return my_pos
            if slot <= R_HOPS:
                return lax.rem(my_pos - slot + N_DEV, N_DEV)
            return lax.rem(my_pos + (slot - R_HOPS), N_DEV)

        def compute(slot):
            head0 = origin_of(slot) * HPB
            wq = wq_comm[slot]
            wo = wo_comm[slot]
            for b in range(B):
                q = jnp.dot(x_ref[b], wq,
                            preferred_element_type=jnp.float32)
                kblk4 = k_ref[b, pl.ds(head0, HPB)]
                vblk4 = v_ref[b, pl.ds(head0, HPB)]
                ctx_parts = []
                for h in range(HPB):
                    qh = (q[:, h * DH:(h + 1) * DH]
                          .astype(jnp.bfloat16).reshape(4, 64, DH))
                    kh = kblk4[h].reshape(4, 64, DH)
                    vh = vblk4[h].reshape(4, 64, DH)
                    scores = lax.dot_general(
                        qh, kh, (((2,), (2,)), ((0,), (0,))),
                        preferred_element_type=jnp.float32,
                    ) * 0.125
                    e = jnp.exp(scores)
                    w = (e / jnp.sum(e, axis=2, keepdims=True)
                         ).astype(jnp.bfloat16)
                    ctx_parts.append(
                        lax.dot_general(
                            w, vh, (((2,), (1,)), ((0,), (0,))),
                            preferred_element_type=jnp.float32,
                        ).reshape(S, DH))
                ctx = jnp.concatenate(ctx_parts, axis=1)
                contrib = jnp.dot(ctx.astype(jnp.bfloat16), wo,
                                  preferred_element_type=jnp.float32)
                if slot == 0:
                    out_ref[b] = contrib
                else:
                    out_ref[b] = out_ref[b] + contrib

        def rdma_pair(src_slot, dst_slot, dev):
            rq = pltpu.make_async_remote_copy(
                src_ref=wq_comm.at[src_slot], dst_ref=wq_comm.at[dst_slot],
                send_sem=swq.at[dst_slot], recv_sem=rwq.at[dst_slot],
                device_id=(dev,), device_id_type=pl.DeviceIdType.MESH,
            )
            ro = pltpu.make_async_remote_copy(
                src_ref=wo_comm.at[src_slot], dst_ref=wo_comm.at[dst_slot],
                send_sem=swo.at[dst_slot], recv_sem=rwo.at[dst_slot],
                device_id=(dev,), device_id_type=pl.DeviceIdType.MESH,
            )
            rq.start()
            ro.start()
            return rq, ro

        for t in range(1, L_HOPS + 1):
            started = []
            if t <= R_HOPS:
                started += rdma_pair(t - 1, t, right)
            lsrc = 0 if t == 1 else R_HOPS + (t - 1)
            started += rdma_pair(lsrc, R_HOPS + t, left)
            if t == 1:
                compute(0)
            else:
                compute(t - 1)
                compute(R_HOPS + (t - 1))
            for r in started:
                r.wait()
        compute(R_HOPS + L_HOPS)

    return pl.pallas_call(
        body,
        out_shape=jax.ShapeDtypeStruct((B, S, D_MODEL), jnp.float32),
        in_specs=[pl.BlockSpec(memory_space=pltpu.VMEM)] * 5,
        out_specs=pl.BlockSpec(memory_space=pltpu.VMEM),
        scratch_shapes=[
            pltpu.VMEM((N_DEV, D_MODEL, HB), jnp.bfloat16),
            pltpu.VMEM((N_DEV, HB, D_MODEL), jnp.bfloat16),
            pltpu.SemaphoreType.DMA((N_DEV,)),
            pltpu.SemaphoreType.DMA((N_DEV,)),
            pltpu.SemaphoreType.DMA((N_DEV,)),
            pltpu.SemaphoreType.DMA((N_DEV,)),
        ],
        compiler_params=pltpu.CompilerParams(collective_id=0),
    )(x_b, Wq, K_t, V_t, Wo)


# device time: 43777 ns/iter; 1.0466x vs baseline; 1.0466x over previous
import jax
import jax.numpy as jnp
from jax import lax
from jax.experimental import pallas as pl
from jax.experimental.pallas import tpu as pltpu

N_DEV = 8
B = 2
S = 256
D_MODEL = 512
HPB = 4
DH = 64
HB = HPB * DH

L_HOPS = N_DEV // 2
R_HOPS = N_DEV - 1 - L_HOPS


def kernel(x, Wq, K_ext, V_ext, Wo):
    K_t = jnp.transpose(K_ext, (0, 2, 1, 3)).astype(jnp.bfloat16)
    V_t = jnp.transpose(V_ext, (0, 2, 1, 3)).astype(jnp.bfloat16)
    x_b = x.astype(jnp.bfloat16)

    def body(x_ref, wq_ref, k_ref, v_ref, wo_ref, out_ref,
             wq_comm, wo_comm, swq, rwq, swo, rwo):
        def sigma(v):
            return jnp.where(v < 4, v, 11 - v)

        my_pos = lax.axis_index("i")
        vi = sigma(my_pos)
        left = sigma(lax.rem(vi + N_DEV - 1, N_DEV))
        right = sigma(lax.rem(vi + 1, N_DEV))

        barrier_sem = pltpu.get_barrier_semaphore()
        for nbr in (left, right):
            pl.semaphore_signal(
                barrier_sem, inc=1,
                device_id=(nbr,), device_id_type=pl.DeviceIdType.MESH,
            )
        pl.semaphore_wait(barrier_sem, 2)

        wq_comm[0] = wq_ref[...].astype(jnp.bfloat16)
        wo_comm[0] = wo_ref[...].astype(jnp.bfloat16)

        def origin_of(slot):
            if slot == 0:
                return my_pos
            if slot <= R_HOPS:
                return sigma(lax.rem(vi - slot + N_DEV, N_DEV))
            return sigma(lax.rem(vi + (slot - R_HOPS), N_DEV))

        def compute(slot):
            head0 = origin_of(slot) * HPB
            wq = wq_comm[slot]
            wo = wo_comm[slot]
            for b in range(B):
                q = jnp.dot(x_ref[b], wq,
                            preferred_element_type=jnp.float32)
                kblk4 = k_ref[b, pl.ds(head0, HPB)]
                vblk4 = v_ref[b, pl.ds(head0, HPB)]
                ctx_parts = []
                for h in range(HPB):
                    qh = (q[:, h * DH:(h + 1) * DH]
                          .astype(jnp.bfloat16).reshape(4, 64, DH))
                    kh = kblk4[h].reshape(4, 64, DH)
                    vh = vblk4[h].reshape(4, 64, DH)
                    scores = lax.dot_general(
                        qh, kh, (((2,), (2,)), ((0,), (0,))),
                        preferred_element_type=jnp.float32,
                    ) * 0.125
                    e = jnp.exp(scores)
                    w = (e / jnp.sum(e, axis=2, keepdims=True)
                         ).astype(jnp.bfloat16)
                    ctx_parts.append(
                        lax.dot_general(
                            w, vh, (((2,), (1,)), ((0,), (0,))),
                            preferred_element_type=jnp.float32,
                        ).reshape(S, DH))
                ctx = jnp.concatenate(ctx_parts, axis=1)
                contrib = jnp.dot(ctx.astype(jnp.bfloat16), wo,
                                  preferred_element_type=jnp.float32)
                if slot == 0:
                    out_ref[b] = contrib
                else:
                    out_ref[b] = out_ref[b] + contrib

        def rdma_pair(src_slot, dst_slot, dev):
            rq = pltpu.make_async_remote_copy(
                src_ref=wq_comm.at[src_slot], dst_ref=wq_comm.at[dst_slot],
                send_sem=swq.at[dst_slot], recv_sem=rwq.at[dst_slot],
                device_id=(dev,), device_id_type=pl.DeviceIdType.MESH,
            )
            ro = pltpu.make_async_remote_copy(
                src_ref=wo_comm.at[src_slot], dst_ref=wo_comm.at[dst_slot],
                send_sem=swo.at[dst_slot], recv_sem=rwo.at[dst_slot],
                device_id=(dev,), device_id_type=pl.DeviceIdType.MESH,
            )
            rq.start()
            ro.start()
            return rq, ro

        for t in range(1, L_HOPS + 1):
            started = []
            if t <= R_HOPS:
                started += rdma_pair(t - 1, t, right)
            lsrc = 0 if t == 1 else R_HOPS + (t - 1)
            started += rdma_pair(lsrc, R_HOPS + t, left)
            if t == 1:
                compute(0)
            else:
                compute(t - 1)
                compute(R_HOPS + (t - 1))
            for r in started:
                r.wait()
        compute(R_HOPS + L_HOPS)

    return pl.pallas_call(
        body,
        out_shape=jax.ShapeDtypeStruct((B, S, D_MODEL), jnp.float32),
        in_specs=[pl.BlockSpec(memory_space=pltpu.VMEM)] * 5,
        out_specs=pl.BlockSpec(memory_space=pltpu.VMEM),
        scratch_shapes=[
            pltpu.VMEM((N_DEV, D_MODEL, HB), jnp.bfloat16),
            pltpu.VMEM((N_DEV, HB, D_MODEL), jnp.bfloat16),
            pltpu.SemaphoreType.DMA((N_DEV,)),
            pltpu.SemaphoreType.DMA((N_DEV,)),
            pltpu.SemaphoreType.DMA((N_DEV,)),
            pltpu.SemaphoreType.DMA((N_DEV,)),
        ],
        compiler_params=pltpu.CompilerParams(collective_id=0),
    )(x_b, Wq, K_t, V_t, Wo)
